# baseline (speedup 1.0000x reference)
import functools

import jax
import jax.numpy as jnp
import numpy as np
from jax import lax
from jax.experimental import pallas as pl
from jax.experimental.pallas import tpu as pltpu
from jax.experimental.pallas import tpu_sc as plsc

SEQLEN = 2048
D_MODEL = 768
BATCH = 4
SCALE = float(np.sqrt(float(D_MODEL)))

NC, NS, L = 2, 16, 16
NW = NC * NS
SEQ_PER_W = SEQLEN // NW
CK = 8
NCH = SEQ_PER_W // CK
RPC = BATCH * CK
NV = D_MODEL // L
NBUF = 4


def _position_encoding(seqlen, d_model, times=10000):
    pos = np.arange(seqlen)[:, np.newaxis].astype(np.float64)
    depths = np.arange(d_model)[np.newaxis, :].astype(np.float64)
    depths = 2 * (depths // 2) / d_model
    angle_rates = 1.0 / times ** depths
    angle_rads = pos * angle_rates
    pe = np.zeros((seqlen, d_model), dtype=np.float64)
    pe[:, 0::2] = np.sin(angle_rads)[:, 0::2]
    pe[:, 1::2] = np.cos(angle_rads)[:, 1::2]
    return pe.astype(np.float32)


_PE = _position_encoding(SEQLEN, D_MODEL)

_mesh = plsc.VectorSubcoreMesh(core_axis_name="c", subcore_axis_name="s")


@functools.partial(
    pl.kernel,
    mesh=_mesh,
    out_type=jax.ShapeDtypeStruct((BATCH * SEQLEN, D_MODEL), jnp.float32),
    scratch_types=[
        pltpu.VMEM((NCH, RPC), jnp.int32),
        pltpu.VMEM((NBUF * CK, D_MODEL), jnp.float32),
        pltpu.VMEM((RPC, D_MODEL), jnp.float32),
        pltpu.VMEM((RPC, D_MODEL), jnp.float32),
        pltpu.VMEM((RPC, D_MODEL), jnp.float32),
        pltpu.VMEM((RPC, D_MODEL), jnp.float32),
        pltpu.SemaphoreType.DMA,
        pltpu.SemaphoreType.DMA,
        pltpu.SemaphoreType.DMA,
        pltpu.SemaphoreType.DMA,
        pltpu.SemaphoreType.DMA,
        pltpu.SemaphoreType.DMA,
        pltpu.SemaphoreType.DMA,
        pltpu.SemaphoreType.DMA,
        pltpu.SemaphoreType.DMA,
        pltpu.SemaphoreType.DMA,
        pltpu.SemaphoreType.DMA,
        pltpu.SemaphoreType.DMA,
        pltpu.SemaphoreType.DMA,
    ],
)
def _emb(x_hbm, pe_hbm, table_hbm, out_hbm,
         idx_v, pe_v, bufA, bufB, bufC, bufD,
         si, g0, g1, g2, g3, p0, p1, p2, p3, o0, o1, o2, o3):
    wid = lax.axis_index("s") * NC + lax.axis_index("c")
    s0 = wid * SEQ_PER_W

    bufs = (bufA, bufB, bufC, bufD)
    gsems = (g0, g1, g2, g3)
    psems = (p0, p1, p2, p3)
    osems = (o0, o1, o2, o3)

    def stage_pe(c):
        return pltpu.async_copy(
            pe_hbm.at[pl.ds(s0 + c * CK, CK)],
            pe_v.at[pl.ds((c % NBUF) * CK, CK)], psems[c % NBUF])

    def gather(c):
        return pltpu.async_copy(
            table_hbm.at[idx_v.at[c]], bufs[c % NBUF], gsems[c % NBUF])

    idx_cp = pltpu.async_copy(x_hbm.at[wid], idx_v, si)
    pe_cps = [stage_pe(0), stage_pe(1)]
    idx_cp.wait()
    gathers = [gather(0), gather(1)]
    scatters = [None] * NCH

    for c in range(NCH):
        buf = bufs[c % NBUF]
        if c + 2 < NCH:
            if c >= 2:
                for s in scatters[c - 2]:
                    s.wait()
            gathers.append(gather(c + 2))
            pe_cps.append(stage_pe(c + 2))
        gathers[c].wait()
        pe_cps[c].wait()

        halves = ((0, BATCH // 2), (BATCH // 2, BATCH)) if c == NCH - 1 \
            else ((0, BATCH),)
        scatters[c] = []
        for blo, bhi in halves:
            @plsc.parallel_loop(0, NV, 1)
            def vec_body(j, buf=buf, c=c, blo=blo, bhi=bhi):
                sl = pl.ds(j * L, L)
                for i in range(CK):
                    pv = pe_v[(c % NBUF) * CK + i, sl]
                    for b in range(blo, bhi):
                        buf[b * CK + i, sl] = buf[b * CK + i, sl] * SCALE + pv
            scatters[c] += [
                pltpu.async_copy(
                    buf.at[pl.ds(b * CK, CK)],
                    out_hbm.at[pl.ds(b * SEQLEN + s0 + c * CK, CK)],
                    osems[c % NBUF])
                for b in range(blo, bhi)]

    for c in range(NCH - 4, NCH):
        for s in scatters[c]:
            s.wait()


def kernel(x, table):
    idx = (x.astype(jnp.int32)
           .reshape(BATCH, NW, NCH, CK)
           .transpose(1, 2, 0, 3)
           .reshape(NW, NCH, RPC))
    out = _emb(idx, _PE, table)
    return out.reshape(BATCH, SEQLEN, D_MODEL)


# speedup vs baseline: 1.0333x; 1.0333x over previous
import functools

import jax
import jax.numpy as jnp
import numpy as np
from jax import lax
from jax.experimental import pallas as pl
from jax.experimental.pallas import tpu as pltpu
from jax.experimental.pallas import tpu_sc as plsc

SEQLEN = 2048
D_MODEL = 768
BATCH = 4
SCALE = float(np.sqrt(float(D_MODEL)))

NC, NS, L = 2, 16, 16
NW = NC * NS
SEQ_PER_W = SEQLEN // NW
CK = 8
NCH = SEQ_PER_W // CK
RPC = BATCH * CK
NV = D_MODEL // L
NBUF = 4


def _position_encoding(seqlen, d_model, times=10000):
    pos = np.arange(seqlen)[:, np.newaxis].astype(np.float64)
    depths = np.arange(d_model)[np.newaxis, :].astype(np.float64)
    depths = 2 * (depths // 2) / d_model
    angle_rates = 1.0 / times ** depths
    angle_rads = pos * angle_rates
    pe = np.zeros((seqlen, d_model), dtype=np.float64)
    pe[:, 0::2] = np.sin(angle_rads)[:, 0::2]
    pe[:, 1::2] = np.cos(angle_rads)[:, 1::2]
    return pe.astype(np.float32)


_PE = _position_encoding(SEQLEN, D_MODEL)

_mesh = plsc.VectorSubcoreMesh(core_axis_name="c", subcore_axis_name="s")


@functools.partial(
    pl.kernel,
    mesh=_mesh,
    out_type=jax.ShapeDtypeStruct((BATCH * SEQLEN, D_MODEL), jnp.float32),
    scratch_types=[
        pltpu.VMEM((NCH, RPC), jnp.int32),
        pltpu.VMEM((NBUF * CK, D_MODEL), jnp.float32),
        pltpu.VMEM((RPC, D_MODEL), jnp.float32),
        pltpu.VMEM((RPC, D_MODEL), jnp.float32),
        pltpu.VMEM((RPC, D_MODEL), jnp.float32),
        pltpu.VMEM((RPC, D_MODEL), jnp.float32),
        pltpu.SemaphoreType.DMA,
        pltpu.SemaphoreType.DMA,
        pltpu.SemaphoreType.DMA,
        pltpu.SemaphoreType.DMA,
        pltpu.SemaphoreType.DMA,
        pltpu.SemaphoreType.DMA,
        pltpu.SemaphoreType.DMA,
        pltpu.SemaphoreType.DMA,
        pltpu.SemaphoreType.DMA,
        pltpu.SemaphoreType.DMA,
        pltpu.SemaphoreType.DMA,
        pltpu.SemaphoreType.DMA,
        pltpu.SemaphoreType.DMA,
    ],
)
def _emb(x_hbm, pe_hbm, table_hbm, out_hbm,
         idx_v, pe_v, bufA, bufB, bufC, bufD,
         si, g0, g1, g2, g3, p0, p1, p2, p3, o0, o1, o2, o3):
    wid = lax.axis_index("s") * NC + lax.axis_index("c")
    s0 = wid * SEQ_PER_W

    bufs = (bufA, bufB, bufC, bufD)
    gsems = (g0, g1, g2, g3)
    psems = (p0, p1, p2, p3)
    osems = (o0, o1, o2, o3)

    def stage_idx(c):
        return [pltpu.async_copy(
            x_hbm.at[pl.ds(b * SEQLEN + s0 + c * CK, CK)],
            idx_v.at[c].at[pl.ds(b * CK, CK)], si)
            for b in range(BATCH)]

    def stage_pe(c):
        return pltpu.async_copy(
            pe_hbm.at[pl.ds(s0 + c * CK, CK)],
            pe_v.at[pl.ds((c % NBUF) * CK, CK)], psems[c % NBUF])

    def gather(c):
        return pltpu.async_copy(
            table_hbm.at[idx_v.at[c]], bufs[c % NBUF], gsems[c % NBUF])

    idx_cps = [stage_idx(0), stage_idx(1)]
    pe_cps = [stage_pe(0), stage_pe(1)]
    for cp in idx_cps[0]:
        cp.wait()
    gathers = [gather(0)]
    for cp in idx_cps[1]:
        cp.wait()
    gathers.append(gather(1))
    for c in range(2, NCH):
        idx_cps.append(stage_idx(c))
    scatters = [None] * NCH

    for c in range(NCH):
        buf = bufs[c % NBUF]
        if c + 2 < NCH:
            if c >= 2:
                for s in scatters[c - 2]:
                    s.wait()
            for cp in idx_cps[c + 2]:
                cp.wait()
            gathers.append(gather(c + 2))
            pe_cps.append(stage_pe(c + 2))
        gathers[c].wait()
        pe_cps[c].wait()

        halves = ((0, BATCH // 2), (BATCH // 2, BATCH)) if c == NCH - 1 \
            else ((0, BATCH),)
        scatters[c] = []
        for blo, bhi in halves:
            @plsc.parallel_loop(0, NV, 1)
            def vec_body(j, buf=buf, c=c, blo=blo, bhi=bhi):
                sl = pl.ds(j * L, L)
                for i in range(CK):
                    pv = pe_v[(c % NBUF) * CK + i, sl]
                    for b in range(blo, bhi):
                        buf[b * CK + i, sl] = buf[b * CK + i, sl] * SCALE + pv
            scatters[c] += [
                pltpu.async_copy(
                    buf.at[pl.ds(b * CK, CK)],
                    out_hbm.at[pl.ds(b * SEQLEN + s0 + c * CK, CK)],
                    osems[c % NBUF])
                for b in range(blo, bhi)]

    for c in range(NCH - 4, NCH):
        for s in scatters[c]:
            s.wait()


def kernel(x, table):
    out = _emb(x.astype(jnp.int32).reshape(-1), _PE, table)
    return out.reshape(BATCH, SEQLEN, D_MODEL)
